# baseline (speedup 1.0000x reference)
import functools

import jax
import jax.numpy as jnp
from jax import lax
from jax.experimental import pallas as pl
from jax.experimental.pallas import tpu as pltpu
from jax.experimental.pallas import tpu_sc as plsc

B = 4096
L = 200
D = 8
NC = 2
NS = 16
NW = NC * NS
RPW = B // NW
S1 = 128
S2 = L - S1
NV1 = S1 * D // 16
NV2 = S2 * D // 16
NB = 4

_mesh = plsc.VectorSubcoreMesh(
    core_axis_name="c", subcore_axis_name="s", num_cores=NC, num_subcores=NS
)


@functools.partial(
    pl.kernel,
    mesh=_mesh,
    out_type=(
        jax.ShapeDtypeStruct((B * 16,), jnp.float32),
        jax.ShapeDtypeStruct((B, D), jnp.float32),
    ),
    scratch_types=(
        pltpu.VMEM((RPW * L,), jnp.int32),
        pltpu.VMEM((RPW,), jnp.int32),
        pltpu.VMEM((RPW,), jnp.int32),
        pltpu.VMEM((RPW, D), jnp.float32),
        [pltpu.VMEM((S1, D), jnp.float32) for _ in range(NB)],
        [pltpu.VMEM((S2, D), jnp.float32) for _ in range(NB)],
        pltpu.VMEM((RPW * 16,), jnp.float32),
        [pltpu.SemaphoreType.DMA for _ in range(NB)],
    ),
    compiler_params=pltpu.CompilerParams(
        needs_layout_passes=False, use_tc_tiling_on_sc=False),
)
def _sc_gather_sum(seq_hbm, len_hbm, table_hbm,
                   ms_out, mt_out,
                   idxblk, len_v, lastid_v, lastrow_v,
                   bufAs, bufBs, stage, sems):
    wid = lax.axis_index("s") * NC + lax.axis_index("c")
    base = wid * RPW

    pltpu.sync_copy(seq_hbm.at[pl.ds(base * L, RPW * L)], idxblk)
    pltpu.sync_copy(len_hbm.at[pl.ds(base, RPW)], len_v)

    lane = lax.iota(jnp.int32, 16)

    for k in range(RPW // 16):
        lv = len_v[pl.ds(k * 16, 16)]
        pos = (k * 16 + lane) * L + lv - 1
        lastid_v[pl.ds(k * 16, 16)] = plsc.load_gather(idxblk, [pos])
    pltpu.async_copy(table_hbm.at[lastid_v], lastrow_v, sems[0]).wait()
    pltpu.sync_copy(lastrow_v, mt_out.at[pl.ds(base, RPW)])

    rowpat = lane >> 3
    colpat = lane & 7

    def issue(row, b):
        pltpu.async_copy(
            table_hbm.at[idxblk.at[pl.ds(row * L, S1)]], bufAs[b], sems[b])
        pltpu.async_copy(
            table_hbm.at[idxblk.at[pl.ds(row * L + S1, S2)]], bufBs[b],
            sems[b])

    def wait_set(b):
        pltpu.make_async_copy(
            table_hbm.at[idxblk.at[pl.ds(0, S1)]], bufAs[b], sems[b]).wait()
        pltpu.make_async_copy(
            table_hbm.at[idxblk.at[pl.ds(0, S2)]], bufBs[b], sems[b]).wait()

    for b in range(NB):
        issue(b, b)

    @pl.loop(0, RPW, step=NB)
    def row_loop(r):
        for b in range(NB):
            rr = r + b
            wait_set(b)
            accs = [jnp.zeros((16,), jnp.float32) for _ in range(4)]
            for j in range(NV1):
                accs[j & 3] = accs[j & 3] + plsc.load_gather(
                    bufAs[b], [rowpat + 2 * j, colpat])
            for j in range(NV2):
                accs[j & 3] = accs[j & 3] + plsc.load_gather(
                    bufBs[b], [rowpat + 2 * j, colpat])
            acc = (accs[0] + accs[1]) + (accs[2] + accs[3])
            stage[pl.ds(rr * 16, 16)] = acc
            issue((rr + NB) % RPW, b)

    for b in range(NB):
        wait_set(b)

    pltpu.sync_copy(stage, ms_out.at[pl.ds(base * 16, RPW * 16)])


def _tc_tail(st_ref, lenf_ref, mt_ref, wat_ref, ba_ref, wbt_ref, bb_ref,
             out_ref):
    st = st_ref[...]
    ms = (st[:, :D] + st[:, D:]) / lenf_ref[...]
    hs = jnp.tanh(
        jnp.dot(ms, wat_ref[...], preferred_element_type=jnp.float32)
        + ba_ref[...])
    ht = jnp.tanh(
        jnp.dot(mt_ref[...], wbt_ref[...], preferred_element_type=jnp.float32)
        + bb_ref[...])
    out_ref[...] = hs * ht


_tc_call = pl.pallas_call(
    _tc_tail,
    out_shape=jax.ShapeDtypeStruct((B, D), jnp.float32),
)


def kernel(item_seq, item_seq_len, emb_table, Wa, ba, Wb, bb):
    lens = item_seq_len.astype(jnp.int32)
    seq_flat = item_seq.astype(jnp.int32).reshape(B * L)
    stage_flat, mt = _sc_gather_sum(seq_flat, lens, emb_table)
    stage2d = stage_flat.reshape(B, 16)
    lenf = lens.astype(jnp.float32).reshape(B, 1)
    return _tc_call(stage2d, lenf, mt, Wa.T, ba.reshape(1, D), Wb.T,
                    bb.reshape(1, D))


# speedup vs baseline: 1.0001x; 1.0001x over previous
import functools

import jax
import jax.numpy as jnp
from jax import lax
from jax.experimental import pallas as pl
from jax.experimental.pallas import tpu as pltpu
from jax.experimental.pallas import tpu_sc as plsc

B = 4096
L = 200
D = 8
W = 16
V2 = 500000
NC = 2
NS = 16
NW = NC * NS
RPW = B // NW
S1 = 128
S2 = L - S1
NV1 = S1 // 2
NV2 = S2 // 2
NB = 4

_mesh = plsc.VectorSubcoreMesh(
    core_axis_name="c", subcore_axis_name="s", num_cores=NC, num_subcores=NS
)


@functools.partial(
    pl.kernel,
    mesh=_mesh,
    out_type=(
        jax.ShapeDtypeStruct((B * 16,), jnp.float32),
        jax.ShapeDtypeStruct((B * D,), jnp.float32),
    ),
    scratch_types=(
        pltpu.VMEM((RPW * L,), jnp.int32),
        pltpu.VMEM((RPW * L,), jnp.int32),
        pltpu.VMEM((RPW,), jnp.int32),
        pltpu.VMEM((RPW,), jnp.int32),
        pltpu.VMEM((RPW,), jnp.int32),
        pltpu.VMEM((RPW, W), jnp.float32),
        [pltpu.VMEM((S1, W), jnp.float32) for _ in range(4)],
        [pltpu.VMEM((S2, W), jnp.float32) for _ in range(4)],
        pltpu.VMEM((RPW * 16,), jnp.float32),
        pltpu.VMEM((RPW * D,), jnp.float32),
        [pltpu.SemaphoreType.DMA for _ in range(4)],
    ),
    compiler_params=pltpu.CompilerParams(
        needs_layout_passes=False, use_tc_tiling_on_sc=False),
)
def _sc_gather_sum(seq_hbm, par_hbm, len_hbm, table_hbm,
                   ms_out, mt_out,
                   idxblk, parblk, len_v, lastid_v, lastpar_v, lastrow_v,
                   bufAs, bufBs, stage, mt_stage, sems):
    wid = lax.axis_index("s") * NC + lax.axis_index("c")
    base = wid * RPW

    pltpu.sync_copy(seq_hbm.at[pl.ds(base * L, RPW * L)], idxblk)
    pltpu.sync_copy(par_hbm.at[pl.ds(base * L, RPW * L)], parblk)
    pltpu.sync_copy(len_hbm.at[pl.ds(base, RPW)], len_v)

    lane = lax.iota(jnp.int32, 16)
    rowpat = lane >> 3
    colpat = lane & 7

    for k in range(RPW // 16):
        lv = len_v[pl.ds(k * 16, 16)]
        pos = (k * 16 + lane) * L + lv - 1
        lastid_v[pl.ds(k * 16, 16)] = plsc.load_gather(idxblk, [pos])
        lastpar_v[pl.ds(k * 16, 16)] = plsc.load_gather(parblk, [pos])
    pltpu.async_copy(table_hbm.at[lastid_v.at[pl.ds(0, RPW)]], lastrow_v,
                     sems[0]).wait()
    for m in range(RPW // 2):
        parv = plsc.load_gather(lastpar_v, [rowpat + 2 * m])
        mt_stage[pl.ds(m * 16, 16)] = plsc.load_gather(
            lastrow_v, [rowpat + 2 * m, parv + colpat])
    pltpu.sync_copy(mt_stage, mt_out.at[pl.ds(base * D, RPW * D)])

    def issue(row, b):
        pltpu.async_copy(
            table_hbm.at[idxblk.at[pl.ds(row * L, S1)]], bufAs[b], sems[b])
        pltpu.async_copy(
            table_hbm.at[idxblk.at[pl.ds(row * L + S1, S2)]], bufBs[b],
            sems[b])

    def wait_set(b):
        pltpu.make_async_copy(
            table_hbm.at[idxblk.at[pl.ds(0, S1)]], bufAs[b], sems[b]).wait()
        pltpu.make_async_copy(
            table_hbm.at[idxblk.at[pl.ds(0, S2)]], bufBs[b], sems[b]).wait()

    for b in range(NB):
        issue(b, b)

    @pl.loop(0, RPW, step=NB)
    def row_loop(r):
        for b in range(NB):
            rr = r + b
            wait_set(b)
            pbase = rr * L
            accs = [jnp.zeros((16,), jnp.float32) for _ in range(4)]
            for j in range(NV1):
                parv = plsc.load_gather(parblk, [pbase + rowpat + 2 * j])
                accs[j & 3] = accs[j & 3] + plsc.load_gather(
                    bufAs[b], [rowpat + 2 * j, parv + colpat])
            for j in range(NV2):
                parv = plsc.load_gather(
                    parblk, [pbase + S1 + rowpat + 2 * j])
                accs[j & 3] = accs[j & 3] + plsc.load_gather(
                    bufBs[b], [rowpat + 2 * j, parv + colpat])
            acc = (accs[0] + accs[1]) + (accs[2] + accs[3])
            stage[pl.ds(rr * 16, 16)] = acc
            issue((rr + NB) % RPW, b)

    for b in range(NB):
        wait_set(b)

    pltpu.sync_copy(stage, ms_out.at[pl.ds(base * 16, RPW * 16)])


RB = 4096


def _tc_transpose(tin_ref, out_ref, scr):
    scr[...] = tin_ref[...].T
    pieces = [scr[a::16, :] for a in range(16)]
    out_ref[...] = jnp.concatenate(pieces, axis=1)


_tr_call = pl.pallas_call(
    _tc_transpose,
    grid=(pl.cdiv(1000000, RB),),
    in_specs=[pl.BlockSpec((D, RB), lambda k: (0, k))],
    out_specs=pl.BlockSpec((RB // 16, 128), lambda k: (k, 0)),
    out_shape=jax.ShapeDtypeStruct((62500, 128), jnp.float32),
    scratch_shapes=[pltpu.VMEM((RB, D), jnp.float32)],
)


def _tc_tail(st_ref, lenf_ref, mt_ref, wat_ref, ba_ref, wbt_ref, bb_ref,
             out_ref):
    st = st_ref[...]
    ms = (st[:, :D] + st[:, D:]) / lenf_ref[...]
    hs = jnp.tanh(
        jnp.dot(ms, wat_ref[...], preferred_element_type=jnp.float32)
        + ba_ref[...])
    ht = jnp.tanh(
        jnp.dot(mt_ref[...], wbt_ref[...], preferred_element_type=jnp.float32)
        + bb_ref[...])
    out_ref[...] = hs * ht


_tc_call = pl.pallas_call(
    _tc_tail,
    out_shape=jax.ShapeDtypeStruct((B, D), jnp.float32),
)


def kernel(item_seq, item_seq_len, emb_table, Wa, ba, Wb, bb):
    lens = item_seq_len.astype(jnp.int32)
    seq = item_seq.astype(jnp.int32)
    seq_half = (seq >> 1).reshape(B * L)
    seq_par8 = ((seq & 1) << 3).reshape(B * L)
    table2 = _tr_call(emb_table.T).reshape(V2, W)
    stage_flat, mt_flat = _sc_gather_sum(seq_half, seq_par8, lens, table2)
    stage2d = stage_flat.reshape(B, 16)
    mt = mt_flat.reshape(B, D)
    lenf = lens.astype(jnp.float32).reshape(B, 1)
    return _tc_call(stage2d, lenf, mt, Wa.T, ba.reshape(1, D), Wb.T,
                    bb.reshape(1, D))


# speedup vs baseline: 2.5501x; 2.5499x over previous
import functools

import jax
import jax.numpy as jnp
from jax import lax
from jax.experimental import pallas as pl
from jax.experimental.pallas import tpu as pltpu
from jax.experimental.pallas import tpu_sc as plsc

B = 4096
L = 200
D = 8
VOCAB = 1000000
VPAD = 1048576
W = 16
NC = 2
NS = 16
NW = NC * NS
RPW = B // NW
S1 = 128
S2 = L - S1
NV1 = S1 // 2
NV2 = S2 // 2
NB = 4
CPW = VPAD // NW
CHUNK = 2048

_mesh = plsc.VectorSubcoreMesh(
    core_axis_name="c", subcore_axis_name="s", num_cores=NC, num_subcores=NS
)


RB = 65536


def _tc_linearize(tin_ref, *out_refs):
    x = tin_ref[...]
    for c in range(D):
        out_refs[c][...] = x[c, :]


_lin_call = pl.pallas_call(
    _tc_linearize,
    grid=(VPAD // RB,),
    in_specs=[pl.BlockSpec((D, RB), lambda k: (0, k))],
    out_specs=[pl.BlockSpec((RB,), lambda k: (k,)) for _ in range(D)],
    out_shape=[jax.ShapeDtypeStruct((VPAD,), jnp.float32)
               for _ in range(D)],
)


@functools.partial(
    pl.kernel,
    mesh=_mesh,
    out_type=jax.ShapeDtypeStruct((VPAD * W,), jnp.float32),
    scratch_types=(
        [pltpu.VMEM((CHUNK,), jnp.float32) for _ in range(D)],
        pltpu.VMEM((CHUNK * W,), jnp.float32),
        pltpu.SemaphoreType.DMA,
    ),
    compiler_params=pltpu.CompilerParams(
        needs_layout_passes=False, use_tc_tiling_on_sc=False),
)
def _sc_interleave(c0, c1, c2, c3, c4, c5, c6, c7, tw_out,
                   dimbufs, outbuf, sem):
    cmaj = (c0, c1, c2, c3, c4, c5, c6, c7)
    wid = lax.axis_index("s") * NC + lax.axis_index("c")
    base = wid * CPW
    lane = lax.iota(jnp.int32, 16)

    @pl.loop(0, CPW // CHUNK)
    def chunk_loop(k):
        off = base + CHUNK * k
        copies = [
            pltpu.async_copy(cmaj[c].at[pl.ds(off, CHUNK)], dimbufs[c], sem)
            for c in range(D)
        ]
        for cp in copies:
            cp.wait()

        @pl.loop(0, CHUNK // 16)
        def g_loop(g):
            dest0 = (g * 16 + lane) * W
            for c in range(D):
                v = dimbufs[c][pl.ds(g * 16, 16)]
                plsc.store_scatter(outbuf, [dest0 + c], v)

        pltpu.sync_copy(outbuf, tw_out.at[pl.ds(off * W, CHUNK * W)])


@functools.partial(
    pl.kernel,
    mesh=_mesh,
    out_type=(
        jax.ShapeDtypeStruct((B * 16,), jnp.float32),
        jax.ShapeDtypeStruct((B * D,), jnp.float32),
    ),
    scratch_types=(
        pltpu.VMEM((RPW * L,), jnp.int32),
        pltpu.VMEM((RPW,), jnp.int32),
        pltpu.VMEM((RPW,), jnp.int32),
        pltpu.VMEM((RPW, W), jnp.float32),
        [pltpu.VMEM((S1, W), jnp.float32) for _ in range(NB)],
        [pltpu.VMEM((S2, W), jnp.float32) for _ in range(NB)],
        pltpu.VMEM((RPW * 16,), jnp.float32),
        pltpu.VMEM((RPW * D,), jnp.float32),
        [pltpu.SemaphoreType.DMA for _ in range(NB)],
    ),
    compiler_params=pltpu.CompilerParams(
        needs_layout_passes=False, use_tc_tiling_on_sc=False),
)
def _sc_gather_sum(seq_hbm, len_hbm, table_hbm,
                   ms_out, mt_out,
                   idxblk, len_v, lastid_v, lastrow_v,
                   bufAs, bufBs, stage, mt_stage, sems):
    wid = lax.axis_index("s") * NC + lax.axis_index("c")
    base = wid * RPW

    pltpu.sync_copy(seq_hbm.at[pl.ds(base * L, RPW * L)], idxblk)
    pltpu.sync_copy(len_hbm.at[pl.ds(base, RPW)], len_v)

    lane = lax.iota(jnp.int32, 16)
    rowpat = lane >> 3
    colpat = lane & 7

    for k in range(RPW // 16):
        lv = len_v[pl.ds(k * 16, 16)]
        pos = (k * 16 + lane) * L + lv - 1
        lastid_v[pl.ds(k * 16, 16)] = plsc.load_gather(idxblk, [pos])
    pltpu.async_copy(table_hbm.at[lastid_v.at[pl.ds(0, RPW)]], lastrow_v,
                     sems[0]).wait()
    for m in range(RPW // 2):
        mt_stage[pl.ds(m * 16, 16)] = plsc.load_gather(
            lastrow_v, [rowpat + 2 * m, colpat])
    pltpu.sync_copy(mt_stage, mt_out.at[pl.ds(base * D, RPW * D)])

    def issue(row, b):
        pltpu.async_copy(
            table_hbm.at[idxblk.at[pl.ds(row * L, S1)]], bufAs[b], sems[b])
        pltpu.async_copy(
            table_hbm.at[idxblk.at[pl.ds(row * L + S1, S2)]], bufBs[b],
            sems[b])

    def wait_set(b):
        pltpu.make_async_copy(
            table_hbm.at[idxblk.at[pl.ds(0, S1)]], bufAs[b], sems[b]).wait()
        pltpu.make_async_copy(
            table_hbm.at[idxblk.at[pl.ds(0, S2)]], bufBs[b], sems[b]).wait()

    for b in range(NB):
        issue(b, b)

    @pl.loop(0, RPW, step=NB)
    def row_loop(r):
        for b in range(NB):
            rr = r + b
            wait_set(b)
            accs = [jnp.zeros((16,), jnp.float32) for _ in range(4)]
            for j in range(NV1):
                accs[j & 3] = accs[j & 3] + plsc.load_gather(
                    bufAs[b], [rowpat + 2 * j, colpat])
            for j in range(NV2):
                accs[j & 3] = accs[j & 3] + plsc.load_gather(
                    bufBs[b], [rowpat + 2 * j, colpat])
            acc = (accs[0] + accs[1]) + (accs[2] + accs[3])
            stage[pl.ds(rr * 16, 16)] = acc
            issue((rr + NB) % RPW, b)

    for b in range(NB):
        wait_set(b)

    pltpu.sync_copy(stage, ms_out.at[pl.ds(base * 16, RPW * 16)])


def _tc_tail(st_ref, lenf_ref, mt_ref, wat_ref, ba_ref, wbt_ref, bb_ref,
             out_ref):
    st = st_ref[...]
    ms = (st[:, :D] + st[:, D:]) / lenf_ref[...]
    hs = jnp.tanh(
        jnp.dot(ms, wat_ref[...], preferred_element_type=jnp.float32)
        + ba_ref[...])
    ht = jnp.tanh(
        jnp.dot(mt_ref[...], wbt_ref[...], preferred_element_type=jnp.float32)
        + bb_ref[...])
    out_ref[...] = hs * ht


_tc_call = pl.pallas_call(
    _tc_tail,
    out_shape=jax.ShapeDtypeStruct((B, D), jnp.float32),
)


def kernel(item_seq, item_seq_len, emb_table, Wa, ba, Wb, bb):
    lens = item_seq_len.astype(jnp.int32)
    seq_flat = item_seq.astype(jnp.int32).reshape(B * L)
    cmaj = _lin_call(emb_table.T)
    tablew = _sc_interleave(*cmaj)
    stage_flat, mt_flat = _sc_gather_sum(seq_flat, lens,
                                         tablew.reshape(VPAD, W))
    stage2d = stage_flat.reshape(B, 16)
    mt = mt_flat.reshape(B, D)
    lenf = lens.astype(jnp.float32).reshape(B, 1)
    return _tc_call(stage2d, lenf, mt, Wa.T, ba.reshape(1, D), Wb.T,
                    bb.reshape(1, D))
